# Initial kernel scaffold; baseline (speedup 1.0000x reference)
import jax
import jax.numpy as jnp
from jax.experimental import pallas as pl


def kernel(x, adj_t, W1, b1, W2, b2):
    raise NotImplementedError("write your pallas kernel here")


# speedup vs baseline: 10.6439x; 10.6439x over previous
import jax
import jax.numpy as jnp
from jax import lax
from jax.experimental import pallas as pl
from jax.experimental.pallas import tpu as pltpu
from jax.experimental.pallas import tpu_sc as plsc

N = 10000
D = 128
NC = 2
NS = 16
NW = NC * NS
CHUNK = 96
NPAD = 10240
BR = 1024
ZROWS = 64
ROWS_ACC = NPAD // NS

E = 320000
EPT_CH = -(-E // (NW * CHUNK))
EPT_CH += EPT_CH % 2
EPAD = EPT_CH * NW * CHUNK
EROWS = EPAD // CHUNK


def _deg_body(dst_hbm, deg_hbm, idxv, degl):
    cid = lax.axis_index("c")
    sid = lax.axis_index("s")
    wid = cid * NS + sid
    rpt = EROWS // NW

    def zero(i, c):
        degl[pl.ds(i * 16, 16)] = jnp.zeros((16,), jnp.float32)
        return c
    lax.fori_loop(0, NPAD // 16, zero, 0)

    pltpu.sync_copy(dst_hbm.at[pl.ds(wid * rpt, rpt)], idxv)

    ones = jnp.ones((16,), jnp.float32)

    def row(r, c):
        def col(k, c2):
            idx = idxv[r, pl.ds(k * 16, 16)]
            plsc.addupdate_scatter(degl, [idx], ones)
            return c2
        return lax.fori_loop(0, CHUNK // 16, col, c)
    lax.fori_loop(0, rpt, row, 0)

    pltpu.sync_copy(degl, deg_hbm.at[wid])


def _agg_body(src_hbm, dst_hbm, g_hbm, out_hbm,
              srcv, dstv, buf0, buf1, acc, sem0, sem1):
    cid = lax.axis_index("c")
    sid = lax.axis_index("s")
    wid = cid * NS + sid
    rpt = EROWS // NW

    def zr(r, c):
        def zc(k, c2):
            buf0[r, pl.ds(k * 16, 16)] = jnp.zeros((16,), jnp.float32)
            return c2
        return lax.fori_loop(0, D // 16, zc, c)
    lax.fori_loop(0, ZROWS, zr, 0)

    base = sid * ROWS_ACC

    def za(k, c):
        pltpu.sync_copy(buf0.at[pl.ds(0, ZROWS)],
                        acc.at[pl.ds(base + k * ZROWS, ZROWS)])
        return c
    lax.fori_loop(0, ROWS_ACC // ZROWS, za, 0)

    pltpu.sync_copy(src_hbm.at[pl.ds(wid * rpt, rpt)], srcv)
    pltpu.sync_copy(dst_hbm.at[pl.ds(wid * rpt, rpt)], dstv)

    plsc.subcore_barrier()

    pltpu.async_copy(g_hbm.at[srcv.at[0]], buf0, sem0)

    def step(jj, c):
        j0 = jj * 2
        pltpu.async_copy(g_hbm.at[srcv.at[j0 + 1]], buf1, sem1)
        pltpu.make_async_copy(g_hbm.at[srcv.at[j0]], buf0, sem0).wait()
        pltpu.sync_copy(buf0, acc.at[dstv.at[j0]], add=True)

        @pl.when(jj + 1 < rpt // 2)
        def _():
            pltpu.async_copy(g_hbm.at[srcv.at[j0 + 2]], buf0, sem0)

        pltpu.make_async_copy(g_hbm.at[srcv.at[j0 + 1]], buf1, sem1).wait()
        pltpu.sync_copy(buf1, acc.at[dstv.at[j0 + 1]], add=True)
        return c
    lax.fori_loop(0, rpt // 2, step, 0)

    plsc.subcore_barrier()
    pltpu.sync_copy(acc.at[pl.ds(base, ROWS_ACC)],
                    out_hbm.at[pl.ds(cid * NPAD + base, ROWS_ACC)])


_MESH = plsc.VectorSubcoreMesh(core_axis_name="c", subcore_axis_name="s")

_SC_PARAMS = pltpu.CompilerParams(
    needs_layout_passes=False, use_tc_tiling_on_sc=False
)

_sc_deg = pl.kernel(
    _deg_body,
    out_type=jax.ShapeDtypeStruct((NW, NPAD), jnp.float32),
    mesh=_MESH,
    compiler_params=_SC_PARAMS,
    scratch_types=[
        pltpu.VMEM((EROWS // NW, CHUNK), jnp.int32),
        pltpu.VMEM((NPAD,), jnp.float32),
    ],
)

_sc_agg = pl.kernel(
    _agg_body,
    out_type=jax.ShapeDtypeStruct((NC * NPAD, D), jnp.float32),
    mesh=_MESH,
    compiler_params=_SC_PARAMS,
    scratch_types=[
        pltpu.VMEM((EROWS // NW, CHUNK), jnp.int32),
        pltpu.VMEM((EROWS // NW, CHUNK), jnp.int32),
        pltpu.VMEM((CHUNK, D), jnp.float32),
        pltpu.VMEM((CHUNK, D), jnp.float32),
        pltpu.VMEM_SHARED((NPAD, D), jnp.float32),
        pltpu.SemaphoreType.DMA,
        pltpu.SemaphoreType.DMA,
    ],
)


def _tc1_body(degp_ref, x_ref, w_ref, g_ref, dinv_ref):
    deg = jnp.sum(degp_ref[...], axis=0)
    dinv = lax.rsqrt(1.0 + deg)
    dinv_ref[...] = dinv
    h = jnp.dot(x_ref[...], w_ref[...],
                precision=lax.Precision.HIGHEST,
                preferred_element_type=jnp.float32)
    g_ref[...] = dinv * h


def _tc2_body(p_ref, g1_ref, dinv_ref, b_ref, w_ref, g2_ref):
    dinv = dinv_ref[...]
    agg = p_ref[0] + p_ref[1] + g1_ref[...]
    z = jnp.maximum(dinv * agg + b_ref[...], 0.0)
    h = jnp.dot(z, w_ref[...],
                precision=lax.Precision.HIGHEST,
                preferred_element_type=jnp.float32)
    g2_ref[...] = dinv * h


def _tc3_body(p_ref, g2_ref, dinv_ref, b_ref, out_ref):
    agg = p_ref[0] + p_ref[1] + g2_ref[...]
    out_ref[...] = dinv_ref[...] * agg + b_ref[...]


_GRID = (NPAD // BR,)

_tc1 = pl.pallas_call(
    _tc1_body,
    grid=_GRID,
    in_specs=[
        pl.BlockSpec((NW, BR, 1), lambda i: (0, i, 0)),
        pl.BlockSpec((BR, D), lambda i: (i, 0)),
        pl.BlockSpec((D, D), lambda i: (0, 0)),
    ],
    out_specs=[
        pl.BlockSpec((BR, D), lambda i: (i, 0)),
        pl.BlockSpec((BR, 1), lambda i: (i, 0)),
    ],
    out_shape=[
        jax.ShapeDtypeStruct((NPAD, D), jnp.float32),
        jax.ShapeDtypeStruct((NPAD, 1), jnp.float32),
    ],
)

_tc2 = pl.pallas_call(
    _tc2_body,
    grid=_GRID,
    in_specs=[
        pl.BlockSpec((NC, BR, D), lambda i: (0, i, 0)),
        pl.BlockSpec((BR, D), lambda i: (i, 0)),
        pl.BlockSpec((BR, 1), lambda i: (i, 0)),
        pl.BlockSpec((1, D), lambda i: (0, 0)),
        pl.BlockSpec((D, D), lambda i: (0, 0)),
    ],
    out_specs=pl.BlockSpec((BR, D), lambda i: (i, 0)),
    out_shape=jax.ShapeDtypeStruct((NPAD, D), jnp.float32),
)

_tc3 = pl.pallas_call(
    _tc3_body,
    grid=_GRID,
    in_specs=[
        pl.BlockSpec((NC, BR, D), lambda i: (0, i, 0)),
        pl.BlockSpec((BR, D), lambda i: (i, 0)),
        pl.BlockSpec((BR, 1), lambda i: (i, 0)),
        pl.BlockSpec((1, D), lambda i: (0, 0)),
    ],
    out_specs=pl.BlockSpec((BR, D), lambda i: (i, 0)),
    out_shape=jax.ShapeDtypeStruct((NPAD, D), jnp.float32),
)


@jax.jit
def kernel(x, adj_t, W1, b1, W2, b2):
    x = x.astype(jnp.float32)
    src = adj_t[0].astype(jnp.int32)
    dst = adj_t[1].astype(jnp.int32)
    pad = EPAD - E
    src_p = jnp.concatenate([src, jnp.zeros((pad,), jnp.int32)]).reshape(EROWS, CHUNK)
    dst_p = jnp.concatenate([dst, jnp.full((pad,), N, jnp.int32)]).reshape(EROWS, CHUNK)
    xp = jnp.zeros((NPAD, D), jnp.float32).at[:N].set(x)

    deg = _sc_deg(dst_p).reshape(NW, NPAD, 1)
    g1, dinv = _tc1(deg, xp, W1)
    p1 = _sc_agg(src_p, dst_p, g1).reshape(NC, NPAD, D)
    g2 = _tc2(p1, g1, dinv, b1.reshape(1, D), W2)
    p2 = _sc_agg(src_p, dst_p, g2).reshape(NC, NPAD, D)
    out = _tc3(p2, g2, dinv, b2.reshape(1, D))
    return out[:N]


# speedup vs baseline: 12.4035x; 1.1653x over previous
import jax
import jax.numpy as jnp
from jax import lax
from jax.experimental import pallas as pl
from jax.experimental.pallas import tpu as pltpu
from jax.experimental.pallas import tpu_sc as plsc

N = 10000
D = 128
NC = 2
NS = 16
NW = NC * NS
CHUNK = 96
NPAD = 10240
BR = 1024
ZROWS = 64
ROWS_ACC = NPAD // NS

E = 320000
EPT_CH = -(-E // (NW * CHUNK))
EPT_CH += EPT_CH % 2
EPAD = EPT_CH * NW * CHUNK
EROWS = EPAD // CHUNK


def _deg_body(dst_hbm, deg_hbm, idxv, degl):
    cid = lax.axis_index("c")
    sid = lax.axis_index("s")
    wid = cid * NS + sid
    rpt = EROWS // NW

    def zero(i, c):
        degl[pl.ds(i * 16, 16)] = jnp.zeros((16,), jnp.float32)
        return c
    lax.fori_loop(0, NPAD // 16, zero, 0)

    pltpu.sync_copy(dst_hbm.at[pl.ds(wid * rpt, rpt)], idxv)

    ones = jnp.ones((16,), jnp.float32)

    def row(r, c):
        def col(k, c2):
            idx = idxv[r, pl.ds(k * 16, 16)]
            plsc.addupdate_scatter(degl, [idx], ones)
            return c2
        return lax.fori_loop(0, CHUNK // 16, col, c)
    lax.fori_loop(0, rpt, row, 0)

    pltpu.sync_copy(degl, deg_hbm.at[wid])


def _agg_body(src_hbm, dst_hbm, g_hbm, out_hbm,
              srcv, dstv, buf0, buf1, acc, sem0, sem1):
    cid = lax.axis_index("c")
    sid = lax.axis_index("s")
    wid = cid * NS + sid
    rpt = EROWS // NW

    def zr(r, c):
        def zc(k, c2):
            buf0[r, pl.ds(k * 16, 16)] = jnp.zeros((16,), jnp.float32)
            return c2
        return lax.fori_loop(0, D // 16, zc, c)
    lax.fori_loop(0, ZROWS, zr, 0)

    base = sid * ROWS_ACC

    def za(k, c):
        pltpu.sync_copy(buf0.at[pl.ds(0, ZROWS)],
                        acc.at[pl.ds(base + k * ZROWS, ZROWS)])
        return c
    lax.fori_loop(0, ROWS_ACC // ZROWS, za, 0)

    pltpu.sync_copy(src_hbm.at[pl.ds(wid * rpt, rpt)], srcv)
    pltpu.sync_copy(dst_hbm.at[pl.ds(wid * rpt, rpt)], dstv)

    plsc.subcore_barrier()

    pltpu.async_copy(g_hbm.at[srcv.at[0]], buf0, sem0)

    def step(jj, c):
        j0 = jj * 2
        pltpu.async_copy(g_hbm.at[srcv.at[j0 + 1]], buf1, sem1)
        pltpu.make_async_copy(g_hbm.at[srcv.at[j0]], buf0, sem0).wait()
        pltpu.sync_copy(buf0, acc.at[dstv.at[j0]], add=True)

        @pl.when(jj + 1 < rpt // 2)
        def _():
            pltpu.async_copy(g_hbm.at[srcv.at[j0 + 2]], buf0, sem0)

        pltpu.make_async_copy(g_hbm.at[srcv.at[j0 + 1]], buf1, sem1).wait()
        pltpu.sync_copy(buf1, acc.at[dstv.at[j0 + 1]], add=True)
        return c
    lax.fori_loop(0, rpt // 2, step, 0)

    plsc.subcore_barrier()
    pltpu.sync_copy(acc.at[pl.ds(base, ROWS_ACC)],
                    out_hbm.at[pl.ds(cid * NPAD + base, ROWS_ACC)])


_MESH = plsc.VectorSubcoreMesh(core_axis_name="c", subcore_axis_name="s")

_SC_PARAMS = pltpu.CompilerParams(
    needs_layout_passes=False, use_tc_tiling_on_sc=False
)

_sc_deg = pl.kernel(
    _deg_body,
    out_type=jax.ShapeDtypeStruct((NW, NPAD), jnp.float32),
    mesh=_MESH,
    compiler_params=_SC_PARAMS,
    scratch_types=[
        pltpu.VMEM((EROWS // NW, CHUNK), jnp.int32),
        pltpu.VMEM((NPAD,), jnp.float32),
    ],
)

_sc_agg = pl.kernel(
    _agg_body,
    out_type=jax.ShapeDtypeStruct((NC * NPAD, D), jnp.float32),
    mesh=_MESH,
    compiler_params=_SC_PARAMS,
    scratch_types=[
        pltpu.VMEM((EROWS // NW, CHUNK), jnp.int32),
        pltpu.VMEM((EROWS // NW, CHUNK), jnp.int32),
        pltpu.VMEM((CHUNK, D), jnp.float32),
        pltpu.VMEM((CHUNK, D), jnp.float32),
        pltpu.VMEM_SHARED((NPAD, D), jnp.float32),
        pltpu.SemaphoreType.DMA,
        pltpu.SemaphoreType.DMA,
    ],
)


def _dinv_col(degt):
    s = jnp.dot(degt, jnp.ones((NW, 1), jnp.float32),
                precision=lax.Precision.HIGHEST,
                preferred_element_type=jnp.float32)
    return lax.rsqrt(1.0 + s)


def _tc1_body(degt_ref, x_ref, w_ref, g_ref):
    dinv = _dinv_col(degt_ref[...])
    h = jnp.dot(x_ref[...], w_ref[...],
                precision=lax.Precision.HIGHEST,
                preferred_element_type=jnp.float32)
    g_ref[...] = dinv * h


def _tc2_body(degt_ref, p_ref, g1_ref, b_ref, w_ref, g2_ref):
    dinv = _dinv_col(degt_ref[...])
    agg = p_ref[0] + p_ref[1] + g1_ref[...]
    z = jnp.maximum(dinv * agg + b_ref[...], 0.0)
    h = jnp.dot(z, w_ref[...],
                precision=lax.Precision.HIGHEST,
                preferred_element_type=jnp.float32)
    g2_ref[...] = dinv * h


def _tc3_body(degt_ref, p_ref, g2_ref, b_ref, out_ref):
    dinv = _dinv_col(degt_ref[...])
    agg = p_ref[0] + p_ref[1] + g2_ref[...]
    out_ref[...] = dinv * agg + b_ref[...]


_GRID = (NPAD // BR,)

_DEGT_SPEC = pl.BlockSpec((BR, NW), lambda i: (i, 0))

_tc1 = pl.pallas_call(
    _tc1_body,
    grid=_GRID,
    in_specs=[
        _DEGT_SPEC,
        pl.BlockSpec((BR, D), lambda i: (i, 0)),
        pl.BlockSpec((D, D), lambda i: (0, 0)),
    ],
    out_specs=pl.BlockSpec((BR, D), lambda i: (i, 0)),
    out_shape=jax.ShapeDtypeStruct((NPAD, D), jnp.float32),
)

_tc2 = pl.pallas_call(
    _tc2_body,
    grid=_GRID,
    in_specs=[
        _DEGT_SPEC,
        pl.BlockSpec((NC, BR, D), lambda i: (0, i, 0)),
        pl.BlockSpec((BR, D), lambda i: (i, 0)),
        pl.BlockSpec((1, D), lambda i: (0, 0)),
        pl.BlockSpec((D, D), lambda i: (0, 0)),
    ],
    out_specs=pl.BlockSpec((BR, D), lambda i: (i, 0)),
    out_shape=jax.ShapeDtypeStruct((NPAD, D), jnp.float32),
)

_tc3 = pl.pallas_call(
    _tc3_body,
    grid=_GRID,
    in_specs=[
        _DEGT_SPEC,
        pl.BlockSpec((NC, BR, D), lambda i: (0, i, 0)),
        pl.BlockSpec((BR, D), lambda i: (i, 0)),
        pl.BlockSpec((1, D), lambda i: (0, 0)),
    ],
    out_specs=pl.BlockSpec((BR, D), lambda i: (i, 0)),
    out_shape=jax.ShapeDtypeStruct((NPAD, D), jnp.float32),
)


@jax.jit
def kernel(x, adj_t, W1, b1, W2, b2):
    x = x.astype(jnp.float32)
    src = adj_t[0].astype(jnp.int32)
    dst = adj_t[1].astype(jnp.int32)
    pad = EPAD - E
    pad_dst = N + (jnp.arange(pad, dtype=jnp.int32) % (NPAD - N))
    src_p = jnp.concatenate([src, jnp.zeros((pad,), jnp.int32)]).reshape(EROWS, CHUNK)
    dst_p = jnp.concatenate([dst, pad_dst]).reshape(EROWS, CHUNK)
    xp = jnp.zeros((NPAD, D), jnp.float32).at[:N].set(x)

    degt = _sc_deg(dst_p).T
    g1 = _tc1(degt, xp, W1)
    p1 = _sc_agg(src_p, dst_p, g1).reshape(NC, NPAD, D)
    g2 = _tc2(degt, p1, g1, b1.reshape(1, D), W2)
    p2 = _sc_agg(src_p, dst_p, g2).reshape(NC, NPAD, D)
    out = _tc3(degt, p2, g2, b2.reshape(1, D))
    return out[:N]


# speedup vs baseline: 13.5004x; 1.0884x over previous
import jax
import jax.numpy as jnp
from jax import lax
from jax.experimental import pallas as pl
from jax.experimental.pallas import tpu as pltpu
from jax.experimental.pallas import tpu_sc as plsc

N = 10000
D = 128
NC = 2
NS = 16
NW = NC * NS
CHUNK = 96
NPAD = 10240
BR = 1024
ZROWS = 64
ROWS_ACC = NPAD // NS

E = 320000
EPT_CH = -(-E // (NW * CHUNK))
EPT_CH += EPT_CH % 2
EPAD = EPT_CH * NW * CHUNK
EROWS = EPAD // CHUNK


def _deg_body(dst_hbm, deg_hbm, idxv, degl):
    cid = lax.axis_index("c")
    sid = lax.axis_index("s")
    wid = cid * NS + sid
    rpt = EROWS // NW

    def zero(i, c):
        degl[pl.ds(i * 16, 16)] = jnp.zeros((16,), jnp.float32)
        return c
    lax.fori_loop(0, NPAD // 16, zero, 0)

    pltpu.sync_copy(dst_hbm.at[pl.ds(wid * rpt, rpt)], idxv)

    ones = jnp.ones((16,), jnp.float32)

    def row(r, c):
        def col(k, c2):
            idx = idxv[r, pl.ds(k * 16, 16)]
            plsc.addupdate_scatter(degl, [idx], ones)
            return c2
        return lax.fori_loop(0, CHUNK // 16, col, c)
    lax.fori_loop(0, rpt, row, 0)

    pltpu.sync_copy(degl, deg_hbm.at[wid])


SEG = EROWS // NW
RF = 156
RS = 2 * SEG - RF


def _agg_body(src_hbm, dst_hbm, g_hbm, out_hbm,
              srcv, dstv, buf0, buf1, acc, sem0, sem1):
    cid = lax.axis_index("c")
    sid = lax.axis_index("s")

    def zr(r, c):
        def zc(k, c2):
            buf0[r, pl.ds(k * 16, 16)] = jnp.zeros((16,), jnp.float32)
            return c2
        return lax.fori_loop(0, D // 16, zc, c)
    lax.fori_loop(0, ZROWS, zr, 0)

    base = sid * ROWS_ACC

    def za(k, c):
        pltpu.sync_copy(buf0.at[pl.ds(0, ZROWS)],
                        acc.at[pl.ds(base + k * ZROWS, ZROWS)])
        return c
    lax.fori_loop(0, ROWS_ACC // ZROWS, za, 0)

    plsc.subcore_barrier()

    def run_rows(row0, n):
        pltpu.sync_copy(src_hbm.at[pl.ds(row0, n)], srcv.at[pl.ds(0, n)])
        pltpu.sync_copy(dst_hbm.at[pl.ds(row0, n)], dstv.at[pl.ds(0, n)])
        pltpu.async_copy(g_hbm.at[srcv.at[0]], buf0, sem0)

        def step(jj, c):
            j0 = jj * 2
            pltpu.async_copy(g_hbm.at[srcv.at[j0 + 1]], buf1, sem1)
            pltpu.make_async_copy(g_hbm.at[srcv.at[j0]], buf0, sem0).wait()
            pltpu.sync_copy(buf0, acc.at[dstv.at[j0]], add=True)

            @pl.when(jj + 1 < n // 2)
            def _():
                pltpu.async_copy(g_hbm.at[srcv.at[j0 + 2]], buf0, sem0)

            pltpu.make_async_copy(g_hbm.at[srcv.at[j0 + 1]], buf1, sem1).wait()
            pltpu.sync_copy(buf1, acc.at[dstv.at[j0 + 1]], add=True)
            return c
        lax.fori_loop(0, n // 2, step, 0)

    @pl.when(cid == 0)
    def _():
        run_rows(sid * RF, SEG)
        run_rows(sid * RF + SEG, RF - SEG)

    @pl.when(cid == 1)
    def _():
        run_rows(NS * RF + sid * RS, RS)

    plsc.subcore_barrier()
    pltpu.sync_copy(acc.at[pl.ds(base, ROWS_ACC)],
                    out_hbm.at[pl.ds(cid * NPAD + base, ROWS_ACC)])


_MESH = plsc.VectorSubcoreMesh(core_axis_name="c", subcore_axis_name="s")

_SC_PARAMS = pltpu.CompilerParams(
    needs_layout_passes=False, use_tc_tiling_on_sc=False
)

_sc_deg = pl.kernel(
    _deg_body,
    out_type=jax.ShapeDtypeStruct((NW, NPAD), jnp.float32),
    mesh=_MESH,
    compiler_params=_SC_PARAMS,
    scratch_types=[
        pltpu.VMEM((EROWS // NW, CHUNK), jnp.int32),
        pltpu.VMEM((NPAD,), jnp.float32),
    ],
)

_sc_agg = pl.kernel(
    _agg_body,
    out_type=jax.ShapeDtypeStruct((NC * NPAD, D), jnp.float32),
    mesh=_MESH,
    compiler_params=_SC_PARAMS,
    scratch_types=[
        pltpu.VMEM((EROWS // NW, CHUNK), jnp.int32),
        pltpu.VMEM((EROWS // NW, CHUNK), jnp.int32),
        pltpu.VMEM((CHUNK, D), jnp.float32),
        pltpu.VMEM((CHUNK, D), jnp.float32),
        pltpu.VMEM_SHARED((NPAD, D), jnp.float32),
        pltpu.SemaphoreType.DMA,
        pltpu.SemaphoreType.DMA,
    ],
)


def _dinv_col(degt):
    s = jnp.dot(degt, jnp.ones((NW, 1), jnp.float32),
                precision=lax.Precision.HIGHEST,
                preferred_element_type=jnp.float32)
    return lax.rsqrt(1.0 + s)


def _tc1_body(degt_ref, x_ref, w_ref, g_ref):
    dinv = _dinv_col(degt_ref[...])
    h = jnp.dot(x_ref[...], w_ref[...],
                precision=lax.Precision.HIGHEST,
                preferred_element_type=jnp.float32)
    g_ref[...] = dinv * h


def _tc2_body(degt_ref, p_ref, g1_ref, b_ref, w_ref, g2_ref):
    dinv = _dinv_col(degt_ref[...])
    agg = p_ref[0] + p_ref[1] + g1_ref[...]
    z = jnp.maximum(dinv * agg + b_ref[...], 0.0)
    h = jnp.dot(z, w_ref[...],
                precision=lax.Precision.HIGHEST,
                preferred_element_type=jnp.float32)
    g2_ref[...] = dinv * h


def _tc3_body(degt_ref, p_ref, g2_ref, b_ref, out_ref):
    dinv = _dinv_col(degt_ref[...])
    agg = p_ref[0] + p_ref[1] + g2_ref[...]
    out_ref[...] = dinv * agg + b_ref[...]


_GRID = (NPAD // BR,)

_DEGT_SPEC = pl.BlockSpec((BR, NW), lambda i: (i, 0))

_tc1 = pl.pallas_call(
    _tc1_body,
    grid=_GRID,
    in_specs=[
        _DEGT_SPEC,
        pl.BlockSpec((BR, D), lambda i: (i, 0)),
        pl.BlockSpec((D, D), lambda i: (0, 0)),
    ],
    out_specs=pl.BlockSpec((BR, D), lambda i: (i, 0)),
    out_shape=jax.ShapeDtypeStruct((NPAD, D), jnp.float32),
)

_tc2 = pl.pallas_call(
    _tc2_body,
    grid=_GRID,
    in_specs=[
        _DEGT_SPEC,
        pl.BlockSpec((NC, BR, D), lambda i: (0, i, 0)),
        pl.BlockSpec((BR, D), lambda i: (i, 0)),
        pl.BlockSpec((1, D), lambda i: (0, 0)),
        pl.BlockSpec((D, D), lambda i: (0, 0)),
    ],
    out_specs=pl.BlockSpec((BR, D), lambda i: (i, 0)),
    out_shape=jax.ShapeDtypeStruct((NPAD, D), jnp.float32),
)

_tc3 = pl.pallas_call(
    _tc3_body,
    grid=_GRID,
    in_specs=[
        _DEGT_SPEC,
        pl.BlockSpec((NC, BR, D), lambda i: (0, i, 0)),
        pl.BlockSpec((BR, D), lambda i: (i, 0)),
        pl.BlockSpec((1, D), lambda i: (0, 0)),
    ],
    out_specs=pl.BlockSpec((BR, D), lambda i: (i, 0)),
    out_shape=jax.ShapeDtypeStruct((NPAD, D), jnp.float32),
)


@jax.jit
def kernel(x, adj_t, W1, b1, W2, b2):
    x = x.astype(jnp.float32)
    src = adj_t[0].astype(jnp.int32)
    dst = adj_t[1].astype(jnp.int32)
    pad = EPAD - E
    pad_dst = N + (jnp.arange(pad, dtype=jnp.int32) % (NPAD - N))
    src_p = jnp.concatenate([src, jnp.zeros((pad,), jnp.int32)]).reshape(EROWS, CHUNK)
    dst_p = jnp.concatenate([dst, pad_dst]).reshape(EROWS, CHUNK)
    xp = jnp.zeros((NPAD, D), jnp.float32).at[:N].set(x)

    degt = _sc_deg(dst_p).T
    g1 = _tc1(degt, xp, W1)
    p1 = _sc_agg(src_p, dst_p, g1).reshape(NC, NPAD, D)
    g2 = _tc2(degt, p1, g1, b1.reshape(1, D), W2)
    p2 = _sc_agg(src_p, dst_p, g2).reshape(NC, NPAD, D)
    out = _tc3(degt, p2, g2, b2.reshape(1, D))
    return out[:N]


# speedup vs baseline: 13.6422x; 1.0105x over previous
import jax
import jax.numpy as jnp
from jax import lax
from jax.experimental import pallas as pl
from jax.experimental.pallas import tpu as pltpu
from jax.experimental.pallas import tpu_sc as plsc

N = 10000
D = 128
NC = 2
NS = 16
NW = NC * NS
CHUNK = 96
NPAD = 10240
BR = 1024
ZROWS = 64
ROWS_ACC = NPAD // NS

E = 320000
EPT_CH = -(-E // (NW * CHUNK))
EPT_CH += EPT_CH % 2
EPAD = EPT_CH * NW * CHUNK
EROWS = EPAD // CHUNK


def _deg_body(dst_hbm, deg_hbm, idxv, degl):
    cid = lax.axis_index("c")
    sid = lax.axis_index("s")
    wid = cid * NS + sid
    rpt = EROWS // NW

    def zero(i, c):
        degl[pl.ds(i * 16, 16)] = jnp.zeros((16,), jnp.float32)
        return c
    lax.fori_loop(0, NPAD // 16, zero, 0)

    pltpu.sync_copy(dst_hbm.at[pl.ds(wid * rpt, rpt)], idxv)

    ones = jnp.ones((16,), jnp.float32)

    def row(r, c):
        def col(k, c2):
            idx = idxv[r, pl.ds(k * 16, 16)]
            plsc.addupdate_scatter(degl, [idx], ones)
            return c2
        return lax.fori_loop(0, CHUNK // 16, col, c)
    lax.fori_loop(0, rpt, row, 0)

    pltpu.sync_copy(degl, deg_hbm.at[wid])


SEG = EROWS // NW
RF = 176
RS = 2 * SEG - RF


def _agg_body(src_hbm, dst_hbm, g_hbm, out_hbm,
              srcv, dstv, buf0, buf1, acc, sem0, sem1):
    cid = lax.axis_index("c")
    sid = lax.axis_index("s")

    def zr(r, c):
        def zc(k, c2):
            buf0[r, pl.ds(k * 16, 16)] = jnp.zeros((16,), jnp.float32)
            return c2
        return lax.fori_loop(0, D // 16, zc, c)
    lax.fori_loop(0, ZROWS, zr, 0)

    base = sid * ROWS_ACC

    def za(k, c):
        pltpu.sync_copy(buf0.at[pl.ds(0, ZROWS)],
                        acc.at[pl.ds(base + k * ZROWS, ZROWS)])
        return c
    lax.fori_loop(0, ROWS_ACC // ZROWS, za, 0)

    plsc.subcore_barrier()

    def run_rows(row0, n):
        pltpu.sync_copy(src_hbm.at[pl.ds(row0, n)], srcv.at[pl.ds(0, n)])
        pltpu.sync_copy(dst_hbm.at[pl.ds(row0, n)], dstv.at[pl.ds(0, n)])
        pltpu.async_copy(g_hbm.at[srcv.at[0]], buf0, sem0)

        def step(jj, c):
            j0 = jj * 2
            pltpu.async_copy(g_hbm.at[srcv.at[j0 + 1]], buf1, sem1)
            pltpu.make_async_copy(g_hbm.at[srcv.at[j0]], buf0, sem0).wait()
            pltpu.sync_copy(buf0, acc.at[dstv.at[j0]], add=True)

            @pl.when(jj + 1 < n // 2)
            def _():
                pltpu.async_copy(g_hbm.at[srcv.at[j0 + 2]], buf0, sem0)

            pltpu.make_async_copy(g_hbm.at[srcv.at[j0 + 1]], buf1, sem1).wait()
            pltpu.sync_copy(buf1, acc.at[dstv.at[j0 + 1]], add=True)
            return c
        lax.fori_loop(0, n // 2, step, 0)

    @pl.when(cid == 0)
    def _():
        run_rows(sid * RF, SEG)
        run_rows(sid * RF + SEG, RF - SEG)

    @pl.when(cid == 1)
    def _():
        run_rows(NS * RF + sid * RS, RS)

    plsc.subcore_barrier()
    pltpu.sync_copy(acc.at[pl.ds(base, ROWS_ACC)],
                    out_hbm.at[pl.ds(cid * NPAD + base, ROWS_ACC)])


_MESH = plsc.VectorSubcoreMesh(core_axis_name="c", subcore_axis_name="s")

_SC_PARAMS = pltpu.CompilerParams(
    needs_layout_passes=False, use_tc_tiling_on_sc=False
)

_sc_deg = pl.kernel(
    _deg_body,
    out_type=jax.ShapeDtypeStruct((NW, NPAD), jnp.float32),
    mesh=_MESH,
    compiler_params=_SC_PARAMS,
    scratch_types=[
        pltpu.VMEM((EROWS // NW, CHUNK), jnp.int32),
        pltpu.VMEM((NPAD,), jnp.float32),
    ],
)

_sc_agg = pl.kernel(
    _agg_body,
    out_type=jax.ShapeDtypeStruct((NC * NPAD, D), jnp.float32),
    mesh=_MESH,
    compiler_params=_SC_PARAMS,
    scratch_types=[
        pltpu.VMEM((EROWS // NW, CHUNK), jnp.int32),
        pltpu.VMEM((EROWS // NW, CHUNK), jnp.int32),
        pltpu.VMEM((CHUNK, D), jnp.float32),
        pltpu.VMEM((CHUNK, D), jnp.float32),
        pltpu.VMEM_SHARED((NPAD, D), jnp.float32),
        pltpu.SemaphoreType.DMA,
        pltpu.SemaphoreType.DMA,
    ],
)


def _dinv_col(degt):
    s = jnp.dot(degt, jnp.ones((NW, 1), jnp.float32),
                precision=lax.Precision.HIGHEST,
                preferred_element_type=jnp.float32)
    return lax.rsqrt(1.0 + s)


def _tc1_body(degt_ref, x_ref, w_ref, g_ref):
    dinv = _dinv_col(degt_ref[...])
    h = jnp.dot(x_ref[...], w_ref[...],
                precision=lax.Precision.HIGHEST,
                preferred_element_type=jnp.float32)
    g_ref[...] = dinv * h


def _tc2_body(degt_ref, p_ref, g1_ref, b_ref, w_ref, g2_ref):
    dinv = _dinv_col(degt_ref[...])
    agg = p_ref[0] + p_ref[1] + g1_ref[...]
    z = jnp.maximum(dinv * agg + b_ref[...], 0.0)
    h = jnp.dot(z, w_ref[...],
                precision=lax.Precision.HIGHEST,
                preferred_element_type=jnp.float32)
    g2_ref[...] = dinv * h


def _tc3_body(degt_ref, p_ref, g2_ref, b_ref, out_ref):
    dinv = _dinv_col(degt_ref[...])
    agg = p_ref[0] + p_ref[1] + g2_ref[...]
    out_ref[...] = dinv * agg + b_ref[...]


_GRID = (NPAD // BR,)

_DEGT_SPEC = pl.BlockSpec((BR, NW), lambda i: (i, 0))

_tc1 = pl.pallas_call(
    _tc1_body,
    grid=_GRID,
    in_specs=[
        _DEGT_SPEC,
        pl.BlockSpec((BR, D), lambda i: (i, 0)),
        pl.BlockSpec((D, D), lambda i: (0, 0)),
    ],
    out_specs=pl.BlockSpec((BR, D), lambda i: (i, 0)),
    out_shape=jax.ShapeDtypeStruct((NPAD, D), jnp.float32),
)

_tc2 = pl.pallas_call(
    _tc2_body,
    grid=_GRID,
    in_specs=[
        _DEGT_SPEC,
        pl.BlockSpec((NC, BR, D), lambda i: (0, i, 0)),
        pl.BlockSpec((BR, D), lambda i: (i, 0)),
        pl.BlockSpec((1, D), lambda i: (0, 0)),
        pl.BlockSpec((D, D), lambda i: (0, 0)),
    ],
    out_specs=pl.BlockSpec((BR, D), lambda i: (i, 0)),
    out_shape=jax.ShapeDtypeStruct((NPAD, D), jnp.float32),
)

_tc3 = pl.pallas_call(
    _tc3_body,
    grid=_GRID,
    in_specs=[
        _DEGT_SPEC,
        pl.BlockSpec((NC, BR, D), lambda i: (0, i, 0)),
        pl.BlockSpec((BR, D), lambda i: (i, 0)),
        pl.BlockSpec((1, D), lambda i: (0, 0)),
    ],
    out_specs=pl.BlockSpec((BR, D), lambda i: (i, 0)),
    out_shape=jax.ShapeDtypeStruct((NPAD, D), jnp.float32),
)


@jax.jit
def kernel(x, adj_t, W1, b1, W2, b2):
    x = x.astype(jnp.float32)
    src = adj_t[0].astype(jnp.int32)
    dst = adj_t[1].astype(jnp.int32)
    pad = EPAD - E
    pad_dst = N + (jnp.arange(pad, dtype=jnp.int32) % (NPAD - N))
    src_p = jnp.concatenate([src, jnp.zeros((pad,), jnp.int32)]).reshape(EROWS, CHUNK)
    dst_p = jnp.concatenate([dst, pad_dst]).reshape(EROWS, CHUNK)
    xp = jnp.zeros((NPAD, D), jnp.float32).at[:N].set(x)

    degt = _sc_deg(dst_p).T
    g1 = _tc1(degt, xp, W1)
    p1 = _sc_agg(src_p, dst_p, g1).reshape(NC, NPAD, D)
    g2 = _tc2(degt, p1, g1, b1.reshape(1, D), W2)
    p2 = _sc_agg(src_p, dst_p, g2).reshape(NC, NPAD, D)
    out = _tc3(degt, p2, g2, b2.reshape(1, D))
    return out[:N]
